# baseline (speedup 1.0000x reference)
import functools

import jax
import jax.numpy as jnp
from jax import lax
from jax.experimental import pallas as pl
from jax.experimental.pallas import tpu as pltpu
from jax.experimental.pallas import tpu_sc as plsc

NUM_CLASSES = 1000000
EMB_DIM = 64
BATCH = 16384

_info = plsc.get_sparse_core_info()
_NC, _NS = _info.num_cores, _info.num_subcores
_NW = _NC * _NS
_B_PER_W = BATCH // _NW
_W = 512
_WSH = 9
_SPLIT = 409600
_NWIN = _SPLIT // _W
_MY_NWIN = _NWIN // _NW
_WCAP = 784
_SENT = 1 << 29
_NBUF = 3


def _make_stream_gather():
    mesh = plsc.VectorSubcoreMesh(core_axis_name="c", subcore_axis_name="s")

    @functools.partial(
        pl.kernel,
        mesh=mesh,
        out_type=jax.ShapeDtypeStruct((BATCH, EMB_DIM), jnp.float32),
        scratch_types=[
            pltpu.VMEM((BATCH,), jnp.int32),
            pltpu.VMEM((_WCAP + 16,), jnp.int32),
            pltpu.VMEM((_WCAP + 16,), jnp.int32),
            pltpu.VMEM((EMB_DIM, _W), jnp.float32),
            pltpu.VMEM((EMB_DIM, _W), jnp.float32),
            pltpu.VMEM((EMB_DIM, _W), jnp.float32),
            pltpu.VMEM((16, EMB_DIM), jnp.float32),
            pltpu.SemaphoreType.DMA,
            pltpu.SemaphoreType.DMA,
            pltpu.SemaphoreType.DMA,
            pltpu.SemaphoreType.DMA,
        ],
        compiler_params=pltpu.CompilerParams(needs_layout_passes=False),
    )
    def gather_k(tblt_hbm, lab_hbm, out_hbm,
                 lab_v, wl_lab, wl_pos, buf0, buf1, buf2, rowbuf,
                 wsem0, wsem1, wsem2, rsem):
        bufs = (buf0, buf1, buf2)
        wsems = (wsem0, wsem1, wsem2)
        wid = lax.axis_index("s") * _NC + lax.axis_index("c")
        pltpu.sync_copy(lab_hbm, lab_v)

        sent_vec = jnp.full((16,), _SENT, jnp.int32)
        for i in range((_WCAP + 16) // 16):
            wl_lab[pl.ds(i * 16, 16)] = sent_vec

        def scan_body(g, cnt):
            lv = lab_v[pl.ds(g * 16, 16)]
            win = lax.shift_right_logical(lv, _WSH)
            mine = ((win & jnp.int32(_NW - 1)) == wid) & (lv < _SPLIT)
            pv = lax.iota(jnp.int32, 16) + g * 16
            plsc.store_compressed(wl_lab.at[pl.ds(cnt, 16)], lv, mask=mine)
            plsc.store_compressed(wl_pos.at[pl.ds(cnt, 16)], pv, mask=mine)
            nm = plsc.all_reduce_population_count(mine)
            return jnp.minimum(cnt + nm[0], jnp.int32(_WCAP))

        lax.fori_loop(0, BATCH // 16, scan_body, jnp.int32(0))

        dvecs = [lax.iota(jnp.int32, 16) + q * 16 for q in range(EMB_DIM // 16)]

        def start_win(s, b):
            w = s * _NW + wid
            off = pl.multiple_of(w * _W, 128)
            pltpu.async_copy(tblt_hbm.at[:, pl.ds(off, _W)], bufs[b], wsems[b])

        def wait_win(s, b):
            w = s * _NW + wid
            off = pl.multiple_of(w * _W, 128)
            pltpu.make_async_copy(
                tblt_hbm.at[:, pl.ds(off, _W)], bufs[b], wsems[b]).wait()

        def process_win(s, b):
            w = s * _NW + wid
            off = pl.multiple_of(w * _W, 128)
            win_v = bufs[b]

            def group_body(g, _):
                lv = wl_lab[pl.ds(g * 16, 16)]
                pv = wl_pos[pl.ds(g * 16, 16)]
                m = lax.shift_right_logical(lv, _WSH) == w
                nm = plsc.all_reduce_population_count(m)

                @pl.when(nm[0] > 0)
                def _():
                    mi = jnp.where(m, jnp.int32(1), jnp.int32(0))
                    for k in range(16):
                        @pl.when(mi[k] == 1)
                        def _():
                            c = lv[k] - off
                            bpos = pv[k]
                            cvec = jnp.broadcast_to(c, (16,))
                            for q in range(EMB_DIM // 16):
                                val = plsc.load_gather(
                                    win_v, [dvecs[q], cvec])
                                rowbuf[k, pl.ds(q * 16, 16)] = val
                            pltpu.async_copy(
                                rowbuf.at[k], out_hbm.at[bpos], rsem)
                    for k in range(16):
                        @pl.when(mi[k] == 1)
                        def _():
                            pltpu.make_async_copy(
                                rowbuf.at[k], out_hbm.at[pv[k]], rsem).wait()
                return None

            lax.fori_loop(0, (_WCAP + 16) // 16, group_body, None)

        for b in range(_NBUF):
            start_win(jnp.int32(b), b)

        def outer_body(s0, _):
            for b in range(_NBUF):
                s = s0 * _NBUF + b

                @pl.when(s < _MY_NWIN)
                def _():
                    wait_win(s, b)
                    process_win(s, b)

                    @pl.when(s + _NBUF < _MY_NWIN)
                    def _():
                        start_win(s + _NBUF, b)
            return None

        lax.fori_loop(0, -(-_MY_NWIN // _NBUF), outer_body, None)

    return gather_k


def _make_row_gather():
    mesh = plsc.VectorSubcoreMesh(core_axis_name="c", subcore_axis_name="s")

    @functools.partial(
        pl.kernel,
        mesh=mesh,
        out_type=jax.ShapeDtypeStruct((BATCH, EMB_DIM), jnp.float32),
        scratch_types=[
            pltpu.VMEM((_B_PER_W,), jnp.int32),
            pltpu.VMEM((_B_PER_W, EMB_DIM), jnp.float32),
            pltpu.SemaphoreType.DMA,
        ],
        compiler_params=pltpu.CompilerParams(needs_layout_passes=False),
    )
    def row_k(rm_hbm, lab_hbm, out_hbm, lab_v, rows_v, sem):
        wid = lax.axis_index("s") * _NC + lax.axis_index("c")
        base = wid * _B_PER_W
        pltpu.sync_copy(lab_hbm.at[wid], lab_v)

        def group_body(g, _):
            r0 = g * 16
            lv = lab_v[pl.ds(r0, 16)]
            for k in range(16):
                pltpu.async_copy(rm_hbm.at[lv[k]], rows_v.at[r0 + k], sem)
            return None

        lax.fori_loop(0, _B_PER_W // 16, group_body, None)
        pltpu.make_async_copy(
            rm_hbm.at[pl.ds(0, _B_PER_W)], rows_v, sem).wait()
        pltpu.sync_copy(rows_v, out_hbm.at[pl.ds(base, _B_PER_W)])

    return row_k


_stream_gather = _make_stream_gather()
_row_gather = _make_row_gather()

_BLK = 2048


def _mlp_body(x_ref, w1_ref, b1_ref, w2_ref, b2_ref, o_ref):
    x = x_ref[...]
    h = jnp.dot(x, w1_ref[...], preferred_element_type=jnp.float32) + b1_ref[...]
    h = h * jax.nn.sigmoid(h)
    o_ref[...] = (
        jnp.dot(h, w2_ref[...], preferred_element_type=jnp.float32) + b2_ref[...]
    )


def _mlp(x, W1, b1, W2, b2):
    grid = (BATCH // _BLK,)
    return pl.pallas_call(
        _mlp_body,
        grid=grid,
        in_specs=[
            pl.BlockSpec((_BLK, EMB_DIM), lambda i: (i, 0)),
            pl.BlockSpec((EMB_DIM, EMB_DIM), lambda i: (0, 0)),
            pl.BlockSpec((1, EMB_DIM), lambda i: (0, 0)),
            pl.BlockSpec((EMB_DIM, EMB_DIM), lambda i: (0, 0)),
            pl.BlockSpec((1, EMB_DIM), lambda i: (0, 0)),
        ],
        out_specs=pl.BlockSpec((_BLK, EMB_DIM), lambda i: (i, 0)),
        out_shape=jax.ShapeDtypeStruct((BATCH, EMB_DIM), jnp.float32),
    )(x, W1, b1, W2, b2)


def kernel(label, emb_table, W1, b1, W2, b2):
    lab = label.astype(jnp.int32)
    tblT = jnp.transpose(emb_table)
    out1 = _stream_gather(tblT, lab)

    rm = lax.slice(emb_table, (_SPLIT, 0), (NUM_CLASSES, EMB_DIM))
    lab_hi = jnp.maximum(lab - _SPLIT, 0).reshape(_NW, _B_PER_W)
    out2 = _row_gather(rm, lab_hi)

    emb = jnp.where((lab >= _SPLIT)[:, None], out2, out1)
    return _mlp(emb, W1, b1.reshape(1, EMB_DIM), W2, b2.reshape(1, EMB_DIM))


# speedup vs baseline: 2.6013x; 2.6013x over previous
import functools

import jax
import jax.numpy as jnp
from jax import lax
from jax.experimental import pallas as pl
from jax.experimental.pallas import tpu as pltpu
from jax.experimental.pallas import tpu_sc as plsc

NUM_CLASSES = 1000000
EMB_DIM = 64
BATCH = 16384

_info = plsc.get_sparse_core_info()
_NC, _NS = _info.num_cores, _info.num_subcores
_NW = _NC * _NS
_B_PER_W = BATCH // _NW
_W = 512
_WSH = 9
_SPLIT = 573440
_NWIN = _SPLIT // _W
_MY_NWIN = _NWIN // _NW
_WCAP = 784
_SENT = 1 << 29
_NBUF = 3


def _make_stream_gather():
    mesh = plsc.VectorSubcoreMesh(core_axis_name="c", subcore_axis_name="s")

    @functools.partial(
        pl.kernel,
        mesh=mesh,
        out_type=jax.ShapeDtypeStruct((BATCH, EMB_DIM), jnp.float32),
        scratch_types=[
            pltpu.VMEM((BATCH,), jnp.int32),
            pltpu.VMEM((_WCAP + 16,), jnp.int32),
            pltpu.VMEM((_WCAP + 16,), jnp.int32),
            pltpu.VMEM((EMB_DIM, _W), jnp.float32),
            pltpu.VMEM((EMB_DIM, _W), jnp.float32),
            pltpu.VMEM((EMB_DIM, _W), jnp.float32),
            pltpu.VMEM((16, EMB_DIM), jnp.float32),
            pltpu.SemaphoreType.DMA,
            pltpu.SemaphoreType.DMA,
            pltpu.SemaphoreType.DMA,
            pltpu.SemaphoreType.DMA,
        ],
        compiler_params=pltpu.CompilerParams(needs_layout_passes=False),
        cost_estimate=pl.CostEstimate(
            flops=0, bytes_accessed=600_000_000, transcendentals=0),
    )
    def gather_k(tblt_hbm, lab_hbm, out_hbm,
                 lab_v, wl_lab, wl_pos, buf0, buf1, buf2, rowbuf,
                 wsem0, wsem1, wsem2, rsem):
        bufs = (buf0, buf1, buf2)
        wsems = (wsem0, wsem1, wsem2)
        wid = lax.axis_index("s") * _NC + lax.axis_index("c")
        pltpu.sync_copy(lab_hbm, lab_v)

        sent_vec = jnp.full((16,), _SENT, jnp.int32)
        for i in range((_WCAP + 16) // 16):
            wl_lab[pl.ds(i * 16, 16)] = sent_vec

        def scan_body(g, cnt):
            lv = lab_v[pl.ds(g * 16, 16)]
            win = lax.shift_right_logical(lv, _WSH)
            mine = ((win & jnp.int32(_NW - 1)) == wid) & (lv < _SPLIT)
            pv = lax.iota(jnp.int32, 16) + g * 16
            plsc.store_compressed(wl_lab.at[pl.ds(cnt, 16)], lv, mask=mine)
            plsc.store_compressed(wl_pos.at[pl.ds(cnt, 16)], pv, mask=mine)
            nm = plsc.all_reduce_population_count(mine)
            return jnp.minimum(cnt + nm[0], jnp.int32(_WCAP))

        lax.fori_loop(0, BATCH // 16, scan_body, jnp.int32(0))

        dvecs = [lax.iota(jnp.int32, 16) + q * 16 for q in range(EMB_DIM // 16)]

        def start_win(s, b):
            w = s * _NW + wid
            off = pl.multiple_of(w * _W, 128)
            pltpu.async_copy(tblt_hbm.at[:, pl.ds(off, _W)], bufs[b], wsems[b])

        def wait_win(s, b):
            w = s * _NW + wid
            off = pl.multiple_of(w * _W, 128)
            pltpu.make_async_copy(
                tblt_hbm.at[:, pl.ds(off, _W)], bufs[b], wsems[b]).wait()

        def process_win(s, b):
            w = s * _NW + wid
            off = pl.multiple_of(w * _W, 128)
            win_v = bufs[b]

            def group_body(g, _):
                lv = wl_lab[pl.ds(g * 16, 16)]
                pv = wl_pos[pl.ds(g * 16, 16)]
                m = lax.shift_right_logical(lv, _WSH) == w
                nm = plsc.all_reduce_population_count(m)

                @pl.when(nm[0] > 0)
                def _():
                    mi = jnp.where(m, jnp.int32(1), jnp.int32(0))
                    for k in range(16):
                        @pl.when(mi[k] == 1)
                        def _():
                            c = lv[k] - off
                            bpos = pv[k]
                            cvec = jnp.broadcast_to(c, (16,))
                            for q in range(EMB_DIM // 16):
                                val = plsc.load_gather(
                                    win_v, [dvecs[q], cvec])
                                rowbuf[k, pl.ds(q * 16, 16)] = val
                            pltpu.async_copy(
                                rowbuf.at[k], out_hbm.at[bpos], rsem)
                    for k in range(16):
                        @pl.when(mi[k] == 1)
                        def _():
                            pltpu.make_async_copy(
                                rowbuf.at[k], out_hbm.at[pv[k]], rsem).wait()
                return None

            lax.fori_loop(0, (_WCAP + 16) // 16, group_body, None)

        for b in range(_NBUF):
            start_win(jnp.int32(b), b)

        def outer_body(s0, _):
            for b in range(_NBUF):
                s = s0 * _NBUF + b

                @pl.when(s < _MY_NWIN)
                def _():
                    wait_win(s, b)
                    process_win(s, b)

                    @pl.when(s + _NBUF < _MY_NWIN)
                    def _():
                        start_win(s + _NBUF, b)
            return None

        lax.fori_loop(0, -(-_MY_NWIN // _NBUF), outer_body, None)

    return gather_k


def _make_row_gather():
    mesh = plsc.VectorSubcoreMesh(core_axis_name="c", subcore_axis_name="s")

    @functools.partial(
        pl.kernel,
        mesh=mesh,
        out_type=jax.ShapeDtypeStruct((BATCH, EMB_DIM), jnp.float32),
        scratch_types=[
            pltpu.VMEM((_B_PER_W,), jnp.int32),
            pltpu.VMEM((_B_PER_W, EMB_DIM), jnp.float32),
            pltpu.SemaphoreType.DMA,
        ],
        compiler_params=pltpu.CompilerParams(needs_layout_passes=False),
    )
    def row_k(rm_hbm, lab_hbm, out_hbm, lab_v, rows_v, sem):
        wid = lax.axis_index("s") * _NC + lax.axis_index("c")
        base = wid * _B_PER_W
        pltpu.sync_copy(lab_hbm.at[wid], lab_v)

        def group_body(g, _):
            r0 = g * 16
            lv = lab_v[pl.ds(r0, 16)]
            for k in range(16):
                pltpu.async_copy(rm_hbm.at[lv[k]], rows_v.at[r0 + k], sem)
            return None

        lax.fori_loop(0, _B_PER_W // 16, group_body, None)
        pltpu.make_async_copy(
            rm_hbm.at[pl.ds(0, _B_PER_W)], rows_v, sem).wait()
        pltpu.sync_copy(rows_v, out_hbm.at[pl.ds(base, _B_PER_W)])

    return row_k


_stream_gather = _make_stream_gather()
_row_gather = _make_row_gather()

_BLK = 2048


def _mlp_body(x_ref, w1_ref, b1_ref, w2_ref, b2_ref, o_ref):
    x = x_ref[...]
    h = jnp.dot(x, w1_ref[...], preferred_element_type=jnp.float32) + b1_ref[...]
    h = h * jax.nn.sigmoid(h)
    o_ref[...] = (
        jnp.dot(h, w2_ref[...], preferred_element_type=jnp.float32) + b2_ref[...]
    )


def _mlp(x, W1, b1, W2, b2):
    grid = (BATCH // _BLK,)
    return pl.pallas_call(
        _mlp_body,
        grid=grid,
        in_specs=[
            pl.BlockSpec((_BLK, EMB_DIM), lambda i: (i, 0)),
            pl.BlockSpec((EMB_DIM, EMB_DIM), lambda i: (0, 0)),
            pl.BlockSpec((1, EMB_DIM), lambda i: (0, 0)),
            pl.BlockSpec((EMB_DIM, EMB_DIM), lambda i: (0, 0)),
            pl.BlockSpec((1, EMB_DIM), lambda i: (0, 0)),
        ],
        out_specs=pl.BlockSpec((_BLK, EMB_DIM), lambda i: (i, 0)),
        out_shape=jax.ShapeDtypeStruct((BATCH, EMB_DIM), jnp.float32),
    )(x, W1, b1, W2, b2)


def kernel(label, emb_table, W1, b1, W2, b2):
    lab = label.astype(jnp.int32)
    tblT = jnp.transpose(emb_table)
    out1 = _stream_gather(tblT, lab)

    rm = lax.slice(emb_table, (_SPLIT, 0), (NUM_CLASSES, EMB_DIM))
    spread = lax.iota(jnp.int32, BATCH) % jnp.int32(NUM_CLASSES - _SPLIT)
    lab_hi = jnp.where(lab >= _SPLIT, lab - _SPLIT, spread)
    lab_hi = lab_hi.reshape(_NW, _B_PER_W)
    out2 = _row_gather(rm, lab_hi)

    emb = jnp.where((lab >= _SPLIT)[:, None], out2, out1)
    return _mlp(emb, W1, b1.reshape(1, EMB_DIM), W2, b2.reshape(1, EMB_DIM))


# speedup vs baseline: 2.7170x; 1.0445x over previous
import functools

import jax
import jax.numpy as jnp
from jax import lax
from jax.experimental import pallas as pl
from jax.experimental.pallas import tpu as pltpu
from jax.experimental.pallas import tpu_sc as plsc

NUM_CLASSES = 1000000
EMB_DIM = 64
BATCH = 16384

_info = plsc.get_sparse_core_info()
_NC, _NS = _info.num_cores, _info.num_subcores
_NW = _NC * _NS
_B_PER_W = BATCH // _NW
_W = 512
_WSH = 9
_SPLIT = 589824
_NWIN = _SPLIT // _W
_MY_NWIN = _NWIN // _NW
_WCAP = 784
_SENT = 1 << 29
_NBUF = 3


def _make_stream_gather():
    mesh = plsc.VectorSubcoreMesh(core_axis_name="c", subcore_axis_name="s")

    @functools.partial(
        pl.kernel,
        mesh=mesh,
        out_type=jax.ShapeDtypeStruct((BATCH, EMB_DIM), jnp.float32),
        scratch_types=[
            pltpu.VMEM((BATCH,), jnp.int32),
            pltpu.VMEM((_WCAP + 16,), jnp.int32),
            pltpu.VMEM((_WCAP + 16,), jnp.int32),
            pltpu.VMEM((EMB_DIM, _W), jnp.float32),
            pltpu.VMEM((EMB_DIM, _W), jnp.float32),
            pltpu.VMEM((EMB_DIM, _W), jnp.float32),
            pltpu.VMEM((16, EMB_DIM), jnp.float32),
            pltpu.SemaphoreType.DMA,
            pltpu.SemaphoreType.DMA,
            pltpu.SemaphoreType.DMA,
            pltpu.SemaphoreType.DMA,
        ],
        compiler_params=pltpu.CompilerParams(needs_layout_passes=False),
        cost_estimate=pl.CostEstimate(
            flops=0, bytes_accessed=600_000_000, transcendentals=0),
    )
    def gather_k(tblt_hbm, lab_hbm, out_hbm,
                 lab_v, wl_lab, wl_pos, buf0, buf1, buf2, rowbuf,
                 wsem0, wsem1, wsem2, rsem):
        bufs = (buf0, buf1, buf2)
        wsems = (wsem0, wsem1, wsem2)
        wid = lax.axis_index("s") * _NC + lax.axis_index("c")
        pltpu.sync_copy(lab_hbm, lab_v)

        sent_vec = jnp.full((16,), _SENT, jnp.int32)
        for i in range((_WCAP + 16) // 16):
            wl_lab[pl.ds(i * 16, 16)] = sent_vec

        def scan_body(g, cnt):
            lv = lab_v[pl.ds(g * 16, 16)]
            win = lax.shift_right_logical(lv, _WSH)
            mine = ((win & jnp.int32(_NW - 1)) == wid) & (lv < _SPLIT)
            pv = lax.iota(jnp.int32, 16) + g * 16
            plsc.store_compressed(wl_lab.at[pl.ds(cnt, 16)], lv, mask=mine)
            plsc.store_compressed(wl_pos.at[pl.ds(cnt, 16)], pv, mask=mine)
            nm = plsc.all_reduce_population_count(mine)
            return jnp.minimum(cnt + nm[0], jnp.int32(_WCAP))

        lax.fori_loop(0, BATCH // 16, scan_body, jnp.int32(0))

        dvecs = [lax.iota(jnp.int32, 16) + q * 16 for q in range(EMB_DIM // 16)]

        def start_win(s, b):
            w = s * _NW + wid
            off = pl.multiple_of(w * _W, 128)
            pltpu.async_copy(tblt_hbm.at[:, pl.ds(off, _W)], bufs[b], wsems[b])

        def wait_win(s, b):
            w = s * _NW + wid
            off = pl.multiple_of(w * _W, 128)
            pltpu.make_async_copy(
                tblt_hbm.at[:, pl.ds(off, _W)], bufs[b], wsems[b]).wait()

        def process_win(s, b):
            w = s * _NW + wid
            off = pl.multiple_of(w * _W, 128)
            win_v = bufs[b]

            def group_body(g, _):
                lv = wl_lab[pl.ds(g * 16, 16)]
                pv = wl_pos[pl.ds(g * 16, 16)]
                m = lax.shift_right_logical(lv, _WSH) == w
                nm = plsc.all_reduce_population_count(m)

                @pl.when(nm[0] > 0)
                def _():
                    mi = jnp.where(m, jnp.int32(1), jnp.int32(0))
                    for k in range(16):
                        @pl.when(mi[k] == 1)
                        def _():
                            c = lv[k] - off
                            bpos = pv[k]
                            cvec = jnp.broadcast_to(c, (16,))
                            for q in range(EMB_DIM // 16):
                                val = plsc.load_gather(
                                    win_v, [dvecs[q], cvec])
                                rowbuf[k, pl.ds(q * 16, 16)] = val
                            pltpu.async_copy(
                                rowbuf.at[k], out_hbm.at[bpos], rsem)
                    for k in range(16):
                        @pl.when(mi[k] == 1)
                        def _():
                            pltpu.make_async_copy(
                                rowbuf.at[k], out_hbm.at[pv[k]], rsem).wait()
                return None

            lax.fori_loop(0, (_WCAP + 16) // 16, group_body, None)

        for b in range(_NBUF):
            start_win(jnp.int32(b), b)

        def outer_body(s0, _):
            for b in range(_NBUF):
                s = s0 * _NBUF + b

                @pl.when(s < _MY_NWIN)
                def _():
                    wait_win(s, b)
                    process_win(s, b)

                    @pl.when(s + _NBUF < _MY_NWIN)
                    def _():
                        start_win(s + _NBUF, b)
            return None

        lax.fori_loop(0, -(-_MY_NWIN // _NBUF), outer_body, None)

    return gather_k


def _make_row_gather():
    mesh = plsc.VectorSubcoreMesh(core_axis_name="c", subcore_axis_name="s")

    @functools.partial(
        pl.kernel,
        mesh=mesh,
        out_type=jax.ShapeDtypeStruct((BATCH, EMB_DIM), jnp.float32),
        scratch_types=[
            pltpu.VMEM((_B_PER_W,), jnp.int32),
            pltpu.VMEM((_B_PER_W, EMB_DIM), jnp.float32),
            pltpu.SemaphoreType.DMA,
        ],
        compiler_params=pltpu.CompilerParams(needs_layout_passes=False),
    )
    def row_k(rm_hbm, lab_hbm, out_hbm, lab_v, rows_v, sem):
        wid = lax.axis_index("s") * _NC + lax.axis_index("c")
        base = wid * _B_PER_W
        pltpu.sync_copy(lab_hbm.at[wid], lab_v)

        def group_body(g, _):
            r0 = g * 16
            lv = lab_v[pl.ds(r0, 16)]
            for k in range(16):
                pltpu.async_copy(rm_hbm.at[lv[k]], rows_v.at[r0 + k], sem)
            return None

        lax.fori_loop(0, _B_PER_W // 16, group_body, None)
        pltpu.make_async_copy(
            rm_hbm.at[pl.ds(0, _B_PER_W)], rows_v, sem).wait()
        pltpu.sync_copy(rows_v, out_hbm.at[pl.ds(base, _B_PER_W)])

    return row_k


_stream_gather = _make_stream_gather()
_row_gather = _make_row_gather()

_BLK = 2048


def _mlp_body(x1_ref, x2_ref, lab_ref, w1_ref, b1_ref, w2_ref, b2_ref, o_ref):
    sel = lab_ref[...] >= _SPLIT
    x = jnp.where(sel, x2_ref[...], x1_ref[...])
    h = jnp.dot(x, w1_ref[...], preferred_element_type=jnp.float32) + b1_ref[...]
    h = h * jax.nn.sigmoid(h)
    o_ref[...] = (
        jnp.dot(h, w2_ref[...], preferred_element_type=jnp.float32) + b2_ref[...]
    )


def _mlp(x1, x2, labc, W1, b1, W2, b2):
    grid = (BATCH // _BLK,)
    return pl.pallas_call(
        _mlp_body,
        grid=grid,
        in_specs=[
            pl.BlockSpec((_BLK, EMB_DIM), lambda i: (i, 0)),
            pl.BlockSpec((_BLK, EMB_DIM), lambda i: (i, 0)),
            pl.BlockSpec((_BLK, 1), lambda i: (i, 0)),
            pl.BlockSpec((EMB_DIM, EMB_DIM), lambda i: (0, 0)),
            pl.BlockSpec((1, EMB_DIM), lambda i: (0, 0)),
            pl.BlockSpec((EMB_DIM, EMB_DIM), lambda i: (0, 0)),
            pl.BlockSpec((1, EMB_DIM), lambda i: (0, 0)),
        ],
        out_specs=pl.BlockSpec((_BLK, EMB_DIM), lambda i: (i, 0)),
        out_shape=jax.ShapeDtypeStruct((BATCH, EMB_DIM), jnp.float32),
    )(x1, x2, labc, W1, b1, W2, b2)


def kernel(label, emb_table, W1, b1, W2, b2):
    lab = label.astype(jnp.int32)
    tblT = jnp.transpose(emb_table)
    out1 = _stream_gather(tblT, lab)

    rm = lax.slice(emb_table, (_SPLIT, 0), (NUM_CLASSES, EMB_DIM))
    spread = lax.iota(jnp.int32, BATCH) % jnp.int32(NUM_CLASSES - _SPLIT)
    lab_hi = jnp.where(lab >= _SPLIT, lab - _SPLIT, spread)
    lab_hi = lab_hi.reshape(_NW, _B_PER_W)
    out2 = _row_gather(rm, lab_hi)

    return _mlp(out1, out2, lab.reshape(BATCH, 1),
                W1, b1.reshape(1, EMB_DIM), W2, b2.reshape(1, EMB_DIM))
